# Initial kernel scaffold; baseline (speedup 1.0000x reference)
import jax
import jax.numpy as jnp
from jax.experimental import pallas as pl


def kernel(positions, cos_cached, sin_cached):
    raise NotImplementedError("write your pallas kernel here")


# speedup vs baseline: 3.1949x; 3.1949x over previous
import functools

import jax
import jax.numpy as jnp
from jax import lax
from jax.experimental import pallas as pl
from jax.experimental.pallas import tpu as pltpu
from jax.experimental.pallas import tpu_sc as plsc

_B = 4
_L = 8192
_DH = 64
_N = _B * _L
_NC = 2
_NS = 16
_NW = _NC * _NS
_CH = 128
_NCH = _N // (_NW * _CH)

_mesh = plsc.VectorSubcoreMesh(core_axis_name="c", subcore_axis_name="s")


@functools.partial(
    pl.kernel,
    mesh=_mesh,
    compiler_params=pltpu.CompilerParams(use_tc_tiling_on_sc=False),
    out_type=(
        jax.ShapeDtypeStruct((_N // _CH, _CH, _DH), jnp.float32),
        jax.ShapeDtypeStruct((_N // _CH, _CH, _DH), jnp.float32),
    ),
    scratch_types=[
        pltpu.VMEM((_NCH, _CH), jnp.int32),
        pltpu.VMEM((_NCH, _CH, _DH), jnp.float32),
        pltpu.SemaphoreType.DMA,
    ],
)
def _rope_gather(pos_hbm, cos_hbm, sin_hbm, cos_out, sin_out,
                 idx_v, rows_v, sem):
    wid = lax.axis_index("s") * _NC + lax.axis_index("c")
    base = wid * _NCH
    pltpu.sync_copy(pos_hbm.at[pl.ds(base, _NCH)], idx_v)

    copies = [
        pltpu.async_copy(cos_hbm.at[idx_v.at[j]], rows_v.at[j], sem)
        for j in range(_NCH)
    ]
    for c in copies:
        c.wait()
    pltpu.sync_copy(rows_v, cos_out.at[pl.ds(base, _NCH)])

    copies = [
        pltpu.async_copy(sin_hbm.at[idx_v.at[j]], rows_v.at[j], sem)
        for j in range(_NCH)
    ]
    for c in copies:
        c.wait()
    pltpu.sync_copy(rows_v, sin_out.at[pl.ds(base, _NCH)])


def kernel(positions, cos_cached, sin_cached):
    b, l = positions.shape
    msl, _, dh = cos_cached.shape
    pos = positions.reshape(_N // _CH, _CH)
    cos_t = cos_cached.reshape(msl, dh)
    sin_t = sin_cached.reshape(msl, dh)
    cos_o, sin_o = _rope_gather(pos, cos_t, sin_t)
    return (cos_o.reshape(b, l, 1, dh), sin_o.reshape(b, l, 1, dh))
